# baseline (speedup 1.0000x reference)
import functools

import jax
import jax.numpy as jnp
from jax import lax
from jax.experimental import pallas as pl
from jax.experimental.pallas import tpu as pltpu
from jax.experimental.pallas import tpu_sc as plsc

N_NODES = 10000
D = 128
ALPHA = 0.2

N_PAD = 10240
NW = 32
CHUNK = 128
NCH0 = 56
NCH1 = 104
NEG = -1e30


def _proj_body(x_ref, w_ref, b_ref, a12_ref, h_ref, hv_ref):
    g = pl.program_id(0)
    xblk = x_ref[...]
    h = lax.dot_general(xblk, w_ref[...], (((1,), (1,)), ((), ())),
                        preferred_element_type=jnp.float32)
    h = h + b_ref[...]
    h_ref[...] = h
    hv = lax.dot_general(a12_ref[...], h, (((1,), (1,)), ((), ())),
                         preferred_element_type=jnp.float32)
    rows = g * 512 + lax.broadcasted_iota(jnp.int32, (2, 512), 1)
    hv_ref[...] = jnp.where(rows < N_NODES, hv, NEG)


def _project(x_pad, W_w, W_b2, a12):
    grid = N_PAD // 512
    return pl.pallas_call(
        _proj_body,
        grid=(grid,),
        in_specs=[
            pl.BlockSpec((512, D), lambda g: (g, 0)),
            pl.BlockSpec((D, D), lambda g: (0, 0)),
            pl.BlockSpec((1, D), lambda g: (0, 0)),
            pl.BlockSpec((2, D), lambda g: (0, 0)),
        ],
        out_specs=[
            pl.BlockSpec((512, D), lambda g: (g, 0)),
            pl.BlockSpec((2, 512), lambda g: (0, g)),
        ],
        out_shape=[
            jax.ShapeDtypeStruct((N_PAD, D), jnp.float32),
            jax.ShapeDtypeStruct((2, N_PAD), jnp.float32),
        ],
    )(x_pad, W_w, W_b2, a12)


def _sc_body(h_hbm, h1_hbm, h2_hbm, src_hbm, dst_hbm,
             hp_out, rs_out,
             sidx_v, didx_v, rows_v, e1_v, e2_v, e_v,
             hp_sh, rs_sh,
             gsem, isem, l1sem, l2sem, ssem, esem):
    cid = lax.axis_index("c")
    sid = lax.axis_index("s")
    n_chunks = jnp.where(cid == 0, NCH0, NCH1)
    ebase = jnp.where(cid == 0, sid * (NCH0 * CHUNK),
                      16 * (NCH0 * CHUNK) + sid * (NCH1 * CHUNK))

    pltpu.sync_copy(src_hbm.at[pl.ds(ebase, CHUNK)], sidx_v.at[0])
    pltpu.sync_copy(dst_hbm.at[pl.ds(ebase, CHUNK)], didx_v.at[0])
    pltpu.async_copy(h_hbm.at[didx_v.at[0]], rows_v.at[0], gsem.at[0])
    pltpu.async_copy(h1_hbm.at[sidx_v.at[0]], e1_v.at[0], l1sem.at[0])
    pltpu.async_copy(h2_hbm.at[didx_v.at[0]], e2_v.at[0], l2sem.at[0])

    def _zero_rows(i, _):
        for k in range(8):
            rows_v[1, i, pl.ds(k * 16, 16)] = jnp.zeros((16,), jnp.float32)
        return 0
    lax.fori_loop(0, CHUNK, _zero_rows, 0)
    for k in range(8):
        e_v[0, pl.ds(k * 16, 16)] = jnp.zeros((16,), jnp.float32)
    slab = N_PAD // 16
    for k in range(slab // CHUNK):
        pltpu.sync_copy(rows_v.at[1], hp_sh.at[pl.ds(sid * slab + k * CHUNK, CHUNK), :])
        pltpu.sync_copy(e_v.at[0, pl.ds(0, CHUNK)],
                        rs_sh.at[pl.ds(sid * slab + k * CHUNK, CHUNK)])
    plsc.subcore_barrier()

    def _when(cond, fn):
        fn() if cond is True else pl.when(cond)(fn)

    def _drain_row_scatter(b2, b4):
        pltpu.make_async_copy(rows_v.at[b2], hp_sh.at[sidx_v.at[b4]],
                              ssem.at[b2]).wait()

    def _drain_e_scatter(b2, b4):
        pltpu.make_async_copy(e_v.at[b2, pl.ds(0, CHUNK)],
                              rs_sh.at[sidx_v.at[b4]], esem.at[b2]).wait()

    def _chunk(c, u, e_guard, r_guard, has_next):
        i2 = u % 2
        _when(e_guard, lambda: _drain_e_scatter(i2, (u + 2) % 4))
        def _pf():
            nbase = ebase + (c + 1) * CHUNK
            pltpu.async_copy(src_hbm.at[pl.ds(nbase, CHUNK)],
                             sidx_v.at[(u + 1) % 4], isem.at[0])
            pltpu.async_copy(dst_hbm.at[pl.ds(nbase, CHUNK)],
                             didx_v.at[(u + 1) % 4], isem.at[1])
        _when(has_next, _pf)
        pltpu.make_async_copy(h1_hbm.at[sidx_v.at[u]], e1_v.at[i2],
                              l1sem.at[i2]).wait()
        pltpu.make_async_copy(h2_hbm.at[didx_v.at[u]], e2_v.at[i2],
                              l2sem.at[i2]).wait()
        for j in range(CHUNK // 16):
            t = e1_v[i2, pl.ds(j * 16, 16)] + e2_v[i2, pl.ds(j * 16, 16)]
            t = jnp.where(t > 0, t, ALPHA * t)
            e_v[i2, pl.ds(j * 16, 16)] = jnp.exp(t)
        _when(r_guard, lambda: _drain_row_scatter(1 - i2, (u + 3) % 4))
        def _go():
            nbase = ebase + (c + 1) * CHUNK
            pltpu.make_async_copy(src_hbm.at[pl.ds(nbase, CHUNK)],
                                  sidx_v.at[(u + 1) % 4], isem.at[0]).wait()
            pltpu.make_async_copy(dst_hbm.at[pl.ds(nbase, CHUNK)],
                                  didx_v.at[(u + 1) % 4], isem.at[1]).wait()
            pltpu.async_copy(h_hbm.at[didx_v.at[(u + 1) % 4]],
                             rows_v.at[1 - i2], gsem.at[1 - i2])
            pltpu.async_copy(h1_hbm.at[sidx_v.at[(u + 1) % 4]],
                             e1_v.at[1 - i2], l1sem.at[1 - i2])
            pltpu.async_copy(h2_hbm.at[didx_v.at[(u + 1) % 4]],
                             e2_v.at[1 - i2], l2sem.at[1 - i2])
        _when(has_next, _go)
        pltpu.make_async_copy(h_hbm.at[didx_v.at[u]], rows_v.at[i2],
                              gsem.at[i2]).wait()
        def _scale(j, _):
            es = e_v[i2, pl.ds(j, 16)][0]
            for k in range(8):
                rows_v[i2, j, pl.ds(k * 16, 16)] = \
                    rows_v[i2, j, pl.ds(k * 16, 16)] * es
            return 0
        lax.fori_loop(0, CHUNK, _scale, 0)
        pltpu.async_copy(rows_v.at[i2], hp_sh.at[sidx_v.at[u]],
                         ssem.at[i2], add=True)
        pltpu.async_copy(e_v.at[i2, pl.ds(0, CHUNK)], rs_sh.at[sidx_v.at[u]],
                         esem.at[i2], add=True)

    n4 = n_chunks // 4
    def _quad(i, _):
        c0 = 4 * i
        _chunk(c0 + 0, 0, i > 0, i > 0, True)
        _chunk(c0 + 1, 1, i > 0, True, True)
        _chunk(c0 + 2, 2, True, True, True)
        _chunk(c0 + 3, 3, True, True, i < n4 - 1)
        return 0
    lax.fori_loop(0, n4, _quad, 0)

    _drain_row_scatter(1, 3)
    _drain_e_scatter(0, 2)
    _drain_e_scatter(1, 3)

    plsc.subcore_barrier()

    pltpu.sync_copy(hp_sh.at[pl.ds(sid * slab, slab), :],
                    hp_out.at[cid, pl.ds(sid * slab, slab), :])
    pltpu.sync_copy(rs_sh.at[pl.ds(sid * slab, slab)],
                    rs_out.at[cid, pl.ds(sid * slab, slab)])


def _sc_aggregate(h, h1, h2, src, dst):
    mesh = plsc.VectorSubcoreMesh(core_axis_name="c", subcore_axis_name="s")
    return pl.kernel(
        _sc_body,
        out_type=[
            jax.ShapeDtypeStruct((2, N_PAD, D), jnp.float32),
            jax.ShapeDtypeStruct((2, N_PAD), jnp.float32),
        ],
        mesh=mesh,
        scratch_types=[
            pltpu.VMEM((4, CHUNK), jnp.int32),
            pltpu.VMEM((4, CHUNK), jnp.int32),
            pltpu.VMEM((2, CHUNK, D), jnp.float32),
            pltpu.VMEM((2, CHUNK), jnp.float32),
            pltpu.VMEM((2, CHUNK), jnp.float32),
            pltpu.VMEM((2, CHUNK + 16), jnp.float32),
            pltpu.VMEM_SHARED((N_PAD, D), jnp.float32),
            pltpu.VMEM_SHARED((N_PAD,), jnp.float32),
            pltpu.SemaphoreType.DMA((2,)),
            pltpu.SemaphoreType.DMA((2,)),
            pltpu.SemaphoreType.DMA((2,)),
            pltpu.SemaphoreType.DMA((2,)),
            pltpu.SemaphoreType.DMA((2,)),
            pltpu.SemaphoreType.DMA((2,)),
        ],
        compiler_params=pltpu.CompilerParams(needs_layout_passes=False),
    )(h, h1, h2, src, dst)


def _fin_body(hp_ref, rs_ref, o_ref):
    hp = hp_ref[0] + hp_ref[1]
    rs = rs_ref[0] + rs_ref[1]
    v = hp / rs[:, None]
    o_ref[...] = jnp.where(v > 0, v, jnp.exp(v) - 1.0)


def _finish(hp, rs):
    grid = N_PAD // 512
    return pl.pallas_call(
        _fin_body,
        grid=(grid,),
        in_specs=[
            pl.BlockSpec((2, 512, D), lambda g: (0, g, 0)),
            pl.BlockSpec((2, 512), lambda g: (0, g)),
        ],
        out_specs=pl.BlockSpec((512, D), lambda g: (g, 0)),
        out_shape=jax.ShapeDtypeStruct((N_PAD, D), jnp.float32),
    )(hp, rs)


@jax.jit
def kernel(input, adj, W_w, W_b, a):
    E = adj.shape[0]
    e_pad = NW * N_PAD
    src = adj[:, 0].astype(jnp.int32)
    dst = adj[:, 1].astype(jnp.int32)
    pad = jnp.full((e_pad - E,), N_NODES, jnp.int32)
    src = jnp.concatenate([src, pad])
    dst = jnp.concatenate([dst, pad])
    x_pad = jnp.pad(input, ((0, N_PAD - N_NODES), (0, 0)))
    a12 = a.reshape(2, D)
    W_b2 = W_b.reshape(1, D)

    h, hv = _project(x_pad, W_w, W_b2, a12)
    hp, rs = _sc_aggregate(h, hv[0], hv[1], src, dst)
    out = _finish(hp, rs)
    return out[:N_NODES]


# speedup vs baseline: 1.1411x; 1.1411x over previous
import functools

import jax
import jax.numpy as jnp
from jax import lax
from jax.experimental import pallas as pl
from jax.experimental.pallas import tpu as pltpu
from jax.experimental.pallas import tpu_sc as plsc

N_NODES = 10000
D = 128
ALPHA = 0.2

N_PAD = 10240
NW = 32
CHUNK = 128
NCH0 = 104
NCH1 = 56
NEG = -1e30


def _proj_body(x_ref, w_ref, b_ref, a12_ref, h_ref, hv_ref):
    g = pl.program_id(0)
    xblk = x_ref[...]
    h = lax.dot_general(xblk, w_ref[...], (((1,), (1,)), ((), ())),
                        preferred_element_type=jnp.float32)
    h = h + b_ref[...]
    h_ref[...] = h
    hv = lax.dot_general(a12_ref[...], h, (((1,), (1,)), ((), ())),
                         preferred_element_type=jnp.float32)
    rows = g * 512 + lax.broadcasted_iota(jnp.int32, (2, 512), 1)
    hv_ref[...] = jnp.where(rows < N_NODES, hv, NEG)


def _project(x_pad, W_w, W_b2, a12):
    grid = N_PAD // 512
    return pl.pallas_call(
        _proj_body,
        grid=(grid,),
        in_specs=[
            pl.BlockSpec((512, D), lambda g: (g, 0)),
            pl.BlockSpec((D, D), lambda g: (0, 0)),
            pl.BlockSpec((1, D), lambda g: (0, 0)),
            pl.BlockSpec((2, D), lambda g: (0, 0)),
        ],
        out_specs=[
            pl.BlockSpec((512, D), lambda g: (g, 0)),
            pl.BlockSpec((2, 512), lambda g: (0, g)),
        ],
        out_shape=[
            jax.ShapeDtypeStruct((N_PAD, D), jnp.float32),
            jax.ShapeDtypeStruct((2, N_PAD), jnp.float32),
        ],
    )(x_pad, W_w, W_b2, a12)


def _sc_body(h_hbm, h1_hbm, h2_hbm, src_hbm, dst_hbm,
             hp_out, rs_out,
             sidx_v, didx_v, rows_v, e1_v, e2_v, e_v,
             hp_sh, rs_sh,
             gsem, isem, l1sem, l2sem, ssem, esem):
    cid = lax.axis_index("c")
    sid = lax.axis_index("s")
    n_chunks = jnp.where(cid == 0, NCH0, NCH1)
    ebase = jnp.where(cid == 0, sid * (NCH0 * CHUNK),
                      16 * (NCH0 * CHUNK) + sid * (NCH1 * CHUNK))

    pltpu.sync_copy(src_hbm.at[pl.ds(ebase, CHUNK)], sidx_v.at[0])
    pltpu.sync_copy(dst_hbm.at[pl.ds(ebase, CHUNK)], didx_v.at[0])
    pltpu.async_copy(h_hbm.at[didx_v.at[0]], rows_v.at[0], gsem.at[0])
    pltpu.async_copy(h1_hbm.at[sidx_v.at[0]], e1_v.at[0], l1sem.at[0])
    pltpu.async_copy(h2_hbm.at[didx_v.at[0]], e2_v.at[0], l2sem.at[0])

    def _zero_rows(i, _):
        for k in range(8):
            rows_v[1, i, pl.ds(k * 16, 16)] = jnp.zeros((16,), jnp.float32)
        return 0
    lax.fori_loop(0, CHUNK, _zero_rows, 0)
    for k in range(8):
        e_v[0, pl.ds(k * 16, 16)] = jnp.zeros((16,), jnp.float32)
    slab = N_PAD // 16
    for k in range(slab // CHUNK):
        pltpu.sync_copy(rows_v.at[1], hp_sh.at[pl.ds(sid * slab + k * CHUNK, CHUNK), :])
        pltpu.sync_copy(e_v.at[0, pl.ds(0, CHUNK)],
                        rs_sh.at[pl.ds(sid * slab + k * CHUNK, CHUNK)])
    plsc.subcore_barrier()

    def _when(cond, fn):
        fn() if cond is True else pl.when(cond)(fn)

    def _drain_row_scatter(b2, b4):
        pltpu.make_async_copy(rows_v.at[b2], hp_sh.at[sidx_v.at[b4]],
                              ssem.at[b2]).wait()

    def _drain_e_scatter(b2, b4):
        pltpu.make_async_copy(e_v.at[b2, pl.ds(0, CHUNK)],
                              rs_sh.at[sidx_v.at[b4]], esem.at[b2]).wait()

    def _chunk(c, u, e_guard, r_guard, has_next):
        i2 = u % 2
        _when(e_guard, lambda: _drain_e_scatter(i2, (u + 2) % 4))
        def _pf():
            nbase = ebase + (c + 1) * CHUNK
            pltpu.async_copy(src_hbm.at[pl.ds(nbase, CHUNK)],
                             sidx_v.at[(u + 1) % 4], isem.at[0])
            pltpu.async_copy(dst_hbm.at[pl.ds(nbase, CHUNK)],
                             didx_v.at[(u + 1) % 4], isem.at[1])
        _when(has_next, _pf)
        pltpu.make_async_copy(h1_hbm.at[sidx_v.at[u]], e1_v.at[i2],
                              l1sem.at[i2]).wait()
        pltpu.make_async_copy(h2_hbm.at[didx_v.at[u]], e2_v.at[i2],
                              l2sem.at[i2]).wait()
        for j in range(CHUNK // 16):
            t = e1_v[i2, pl.ds(j * 16, 16)] + e2_v[i2, pl.ds(j * 16, 16)]
            t = jnp.where(t > 0, t, ALPHA * t)
            e_v[i2, pl.ds(j * 16, 16)] = jnp.exp(t)
        _when(r_guard, lambda: _drain_row_scatter(1 - i2, (u + 3) % 4))
        def _go():
            nbase = ebase + (c + 1) * CHUNK
            pltpu.make_async_copy(src_hbm.at[pl.ds(nbase, CHUNK)],
                                  sidx_v.at[(u + 1) % 4], isem.at[0]).wait()
            pltpu.make_async_copy(dst_hbm.at[pl.ds(nbase, CHUNK)],
                                  didx_v.at[(u + 1) % 4], isem.at[1]).wait()
            pltpu.async_copy(h_hbm.at[didx_v.at[(u + 1) % 4]],
                             rows_v.at[1 - i2], gsem.at[1 - i2])
            pltpu.async_copy(h1_hbm.at[sidx_v.at[(u + 1) % 4]],
                             e1_v.at[1 - i2], l1sem.at[1 - i2])
            pltpu.async_copy(h2_hbm.at[didx_v.at[(u + 1) % 4]],
                             e2_v.at[1 - i2], l2sem.at[1 - i2])
        _when(has_next, _go)
        pltpu.make_async_copy(h_hbm.at[didx_v.at[u]], rows_v.at[i2],
                              gsem.at[i2]).wait()
        def _scale(j, _):
            es = e_v[i2, pl.ds(j, 16)][0]
            for k in range(8):
                rows_v[i2, j, pl.ds(k * 16, 16)] = \
                    rows_v[i2, j, pl.ds(k * 16, 16)] * es
            return 0
        lax.fori_loop(0, CHUNK, _scale, 0)
        pltpu.async_copy(rows_v.at[i2], hp_sh.at[sidx_v.at[u]],
                         ssem.at[i2], add=True)
        pltpu.async_copy(e_v.at[i2, pl.ds(0, CHUNK)], rs_sh.at[sidx_v.at[u]],
                         esem.at[i2], add=True)

    n4 = n_chunks // 4
    def _quad(i, _):
        c0 = 4 * i
        _chunk(c0 + 0, 0, i > 0, i > 0, True)
        _chunk(c0 + 1, 1, i > 0, True, True)
        _chunk(c0 + 2, 2, True, True, True)
        _chunk(c0 + 3, 3, True, True, i < n4 - 1)
        return 0
    lax.fori_loop(0, n4, _quad, 0)

    _drain_row_scatter(1, 3)
    _drain_e_scatter(0, 2)
    _drain_e_scatter(1, 3)

    plsc.subcore_barrier()

    pltpu.sync_copy(hp_sh.at[pl.ds(sid * slab, slab), :],
                    hp_out.at[cid, pl.ds(sid * slab, slab), :])
    pltpu.sync_copy(rs_sh.at[pl.ds(sid * slab, slab)],
                    rs_out.at[cid, pl.ds(sid * slab, slab)])


def _sc_aggregate(h, h1, h2, src, dst):
    mesh = plsc.VectorSubcoreMesh(core_axis_name="c", subcore_axis_name="s")
    return pl.kernel(
        _sc_body,
        out_type=[
            jax.ShapeDtypeStruct((2, N_PAD, D), jnp.float32),
            jax.ShapeDtypeStruct((2, N_PAD), jnp.float32),
        ],
        mesh=mesh,
        scratch_types=[
            pltpu.VMEM((4, CHUNK), jnp.int32),
            pltpu.VMEM((4, CHUNK), jnp.int32),
            pltpu.VMEM((2, CHUNK, D), jnp.float32),
            pltpu.VMEM((2, CHUNK), jnp.float32),
            pltpu.VMEM((2, CHUNK), jnp.float32),
            pltpu.VMEM((2, CHUNK + 16), jnp.float32),
            pltpu.VMEM_SHARED((N_PAD, D), jnp.float32),
            pltpu.VMEM_SHARED((N_PAD,), jnp.float32),
            pltpu.SemaphoreType.DMA((2,)),
            pltpu.SemaphoreType.DMA((2,)),
            pltpu.SemaphoreType.DMA((2,)),
            pltpu.SemaphoreType.DMA((2,)),
            pltpu.SemaphoreType.DMA((2,)),
            pltpu.SemaphoreType.DMA((2,)),
        ],
        compiler_params=pltpu.CompilerParams(needs_layout_passes=False),
    )(h, h1, h2, src, dst)


def _fin_body(hp_ref, rs_ref, o_ref):
    hp = hp_ref[0] + hp_ref[1]
    rs = rs_ref[0] + rs_ref[1]
    v = hp / rs[:, None]
    o_ref[...] = jnp.where(v > 0, v, jnp.exp(v) - 1.0)


def _finish(hp, rs):
    grid = N_PAD // 512
    return pl.pallas_call(
        _fin_body,
        grid=(grid,),
        in_specs=[
            pl.BlockSpec((2, 512, D), lambda g: (0, g, 0)),
            pl.BlockSpec((2, 512), lambda g: (0, g)),
        ],
        out_specs=pl.BlockSpec((512, D), lambda g: (g, 0)),
        out_shape=jax.ShapeDtypeStruct((N_PAD, D), jnp.float32),
    )(hp, rs)


@jax.jit
def kernel(input, adj, W_w, W_b, a):
    E = adj.shape[0]
    e_pad = NW * N_PAD
    src = adj[:, 0].astype(jnp.int32)
    dst = adj[:, 1].astype(jnp.int32)
    pad = jnp.full((e_pad - E,), N_NODES, jnp.int32)
    src = jnp.concatenate([src, pad])
    dst = jnp.concatenate([dst, pad])
    x_pad = jnp.pad(input, ((0, N_PAD - N_NODES), (0, 0)))
    a12 = a.reshape(2, D)
    W_b2 = W_b.reshape(1, D)

    h, hv = _project(x_pad, W_w, W_b2, a12)
    hp, rs = _sc_aggregate(h, hv[0], hv[1], src, dst)
    out = _finish(hp, rs)
    return out[:N_NODES]
